# baseline (speedup 1.0000x reference)
import functools

import jax
import jax.numpy as jnp
from jax import lax
from jax.experimental import pallas as pl
from jax.experimental.pallas import tpu as pltpu
from jax.experimental.pallas import tpu_sc as plsc

_D = 1024
_R = 16
_CW = _R * _D
_NB = 4
_NW = 32
_NC = 2


@functools.lru_cache(maxsize=None)
def _make_sc_kernel(B, S, interpret=False):
    rows_pw = S // _NW
    n_chunks = rows_pw // _R
    T = n_chunks * B
    mesh = plsc.VectorSubcoreMesh(core_axis_name="c", subcore_axis_name="s")

    @functools.partial(
        pl.kernel,
        out_type=jax.ShapeDtypeStruct((B * S * _D,), jnp.float32),
        mesh=mesh,
        interpret=interpret,
        scratch_types=[
            pltpu.VMEM((_NB * _CW,), jnp.float32),
            pltpu.VMEM((2 * _CW,), jnp.float32),
            [pltpu.SemaphoreType.DMA] * _NB,
            [pltpu.SemaphoreType.DMA] * _NB,
            [pltpu.SemaphoreType.DMA] * 2,
        ],
    )
    def sc_add(x_hbm, pos_hbm, out_hbm, xbuf, pbuf, in_sems, out_sems, p_sems):
        wid = lax.axis_index("s") * _NC + lax.axis_index("c")
        row0 = wid * rows_pw

        def load_x(t, i):
            c, b = divmod(t, B)
            off = (b * S + row0 + c * _R) * _D
            return pltpu.async_copy(
                x_hbm.at[pl.ds(off, _CW)], xbuf.at[pl.ds(i * _CW, _CW)], in_sems[i]
            )

        def load_p(c):
            off = (row0 + c * _R) * _D
            return pltpu.async_copy(
                pos_hbm.at[pl.ds(off, _CW)],
                pbuf.at[pl.ds((c % 2) * _CW, _CW)],
                p_sems[c % 2],
            )

        def store_x(t, i):
            c, b = divmod(t, B)
            off = (b * S + row0 + c * _R) * _D
            return pltpu.async_copy(
                xbuf.at[pl.ds(i * _CW, _CW)], out_hbm.at[pl.ds(off, _CW)], out_sems[i]
            )

        pdesc = [None, None]
        pdesc[0] = load_p(0)
        in_desc = [None] * _NB
        for t in range(min(_NB - 1, T)):
            in_desc[t % _NB] = load_x(t, t % _NB)
        out_desc = [None] * _NB

        for t in range(T):
            i = t % _NB
            c, b = divmod(t, B)
            in_desc[i].wait()
            if b == 0:
                pdesc[c % 2].wait()
                if c + 1 < n_chunks:
                    pdesc[(c + 1) % 2] = load_p(c + 1)
            pc = (c % 2) * _CW

            @plsc.parallel_loop(0, _CW, step=16, unroll=8)
            def _(j):
                plsc.addupdate(
                    xbuf.at[pl.ds(i * _CW + j, 16)], pbuf[pl.ds(pc + j, 16)]
                )

            out_desc[i] = store_x(t, i)
            nt = t + _NB - 1
            if nt < T:
                nb = nt % _NB
                if out_desc[nb] is not None:
                    out_desc[nb].wait()
                    out_desc[nb] = None
                in_desc[nb] = load_x(nt, nb)

        for i in range(_NB):
            if out_desc[i] is not None:
                out_desc[i].wait()

    return sc_add


def kernel(x, pos_emb):
    B, S, D = x.shape
    out = _make_sc_kernel(B, S)(x.reshape(-1), pos_emb.reshape(-1))
    return out.reshape(B, S, D)


# speedup vs baseline: 2.9755x; 2.9755x over previous
import functools

import jax
import jax.numpy as jnp
from jax import lax
from jax.experimental import pallas as pl
from jax.experimental.pallas import tpu as pltpu
from jax.experimental.pallas import tpu_sc as plsc

_R = 16
_NB = 4
_NW = 32
_NC = 2
_VPR = None


@functools.lru_cache(maxsize=None)
def _make_sc_kernel(B, S, D):
    rows_pw = S // _NW
    n_chunks = rows_pw // _R
    T = n_chunks * B
    vpr = D // 16
    mesh = plsc.VectorSubcoreMesh(core_axis_name="c", subcore_axis_name="s")

    @functools.partial(
        pl.kernel,
        out_type=jax.ShapeDtypeStruct((B, S, D), jnp.float32),
        mesh=mesh,
        scratch_types=[
            pltpu.VMEM((_NB * _R, D), jnp.float32),
            pltpu.VMEM((2 * _R, D), jnp.float32),
            [pltpu.SemaphoreType.DMA] * _NB,
            [pltpu.SemaphoreType.DMA] * _NB,
            [pltpu.SemaphoreType.DMA] * 2,
        ],
    )
    def sc_add(x_hbm, pos_hbm, out_hbm, xbuf, pbuf, in_sems, out_sems, p_sems):
        wid = lax.axis_index("s") * _NC + lax.axis_index("c")
        row0 = wid * rows_pw

        def load_x(t, i):
            c, b = divmod(t, B)
            return pltpu.async_copy(
                x_hbm.at[b, pl.ds(row0 + c * _R, _R)],
                xbuf.at[pl.ds(i * _R, _R)],
                in_sems[i],
            )

        def load_p(c):
            return pltpu.async_copy(
                pos_hbm.at[pl.ds(row0 + c * _R, _R)],
                pbuf.at[pl.ds((c % 2) * _R, _R)],
                p_sems[c % 2],
            )

        def store_x(t, i):
            c, b = divmod(t, B)
            return pltpu.async_copy(
                xbuf.at[pl.ds(i * _R, _R)],
                out_hbm.at[b, pl.ds(row0 + c * _R, _R)],
                out_sems[i],
            )

        pdesc = [None, None]
        pdesc[0] = load_p(0)
        in_desc = [None] * _NB
        for t in range(min(_NB - 1, T)):
            in_desc[t % _NB] = load_x(t, t % _NB)
        out_desc = [None] * _NB

        for t in range(T):
            i = t % _NB
            c, b = divmod(t, B)
            in_desc[i].wait()
            if b == 0:
                pdesc[c % 2].wait()
                if c + 1 < n_chunks:
                    pdesc[(c + 1) % 2] = load_p(c + 1)
            prow = (c % 2) * _R

            @plsc.parallel_loop(0, _R * vpr, step=1, unroll=8)
            def _(j):
                r = j // vpr
                col = (j % vpr) * 16
                plsc.addupdate(
                    xbuf.at[i * _R + r, pl.ds(col, 16)],
                    pbuf[prow + r, pl.ds(col, 16)],
                )

            out_desc[i] = store_x(t, i)
            nt = t + _NB - 1
            if nt < T:
                nb = nt % _NB
                if out_desc[nb] is not None:
                    out_desc[nb].wait()
                    out_desc[nb] = None
                in_desc[nb] = load_x(nt, nb)

        for i in range(_NB):
            if out_desc[i] is not None:
                out_desc[i].wait()

    return sc_add


def kernel(x, pos_emb):
    B, S, D = x.shape
    return _make_sc_kernel(B, S, D)(x, pos_emb)
